# baseline (speedup 1.0000x reference)
import jax
import jax.numpy as jnp
from jax.experimental import pallas as pl


BN = 2048


def _body(x_ref, w_ref, b_ref, o_ref):
    bias = jnp.reshape(b_ref[...], (b_ref.shape[0], 1))
    o_ref[...] = (
        jnp.dot(w_ref[...], x_ref[...], preferred_element_type=jnp.float32)
        + bias
    )


def kernel(scalar, W, b):
    batch, k = scalar.shape
    n = W.shape[1]
    xT = scalar.T
    wT = W.T
    grid = batch // BN
    outT = pl.pallas_call(
        _body,
        grid=(grid,),
        in_specs=[
            pl.BlockSpec((k, BN), lambda i: (0, i)),
            pl.BlockSpec((n, k), lambda i: (0, 0)),
            pl.BlockSpec((n,), lambda i: (0,)),
        ],
        out_specs=pl.BlockSpec((n, BN), lambda i: (0, i)),
        out_shape=jax.ShapeDtypeStruct((n, batch), jnp.float32),
    )(xT, wT, b)
    return outT.T


# speedup vs baseline: 1.4136x; 1.4136x over previous
import jax
import jax.numpy as jnp
from jax.experimental import pallas as pl


BN = 4096


def _body(x_ref, w_ref, b_ref, o_ref):
    bias = jnp.reshape(b_ref[...], (b_ref.shape[0], 1))
    o_ref[...] = (
        jnp.dot(w_ref[...], x_ref[...], preferred_element_type=jnp.float32)
        + bias
    )


def kernel(scalar, W, b):
    batch, k = scalar.shape
    n = W.shape[1]
    xT = scalar.T
    wT = W.T
    grid = batch // BN
    outT = pl.pallas_call(
        _body,
        grid=(grid,),
        in_specs=[
            pl.BlockSpec((k, BN), lambda i: (0, i)),
            pl.BlockSpec((n, k), lambda i: (0, 0)),
            pl.BlockSpec((n,), lambda i: (0,)),
        ],
        out_specs=pl.BlockSpec((n, BN), lambda i: (0, i)),
        out_shape=jax.ShapeDtypeStruct((n, batch), jnp.float32),
    )(xT, wT, b)
    return outT.T


# speedup vs baseline: 1.7698x; 1.2519x over previous
import jax
import jax.numpy as jnp
from jax.experimental import pallas as pl


BN = 8192


def _body(x_ref, w_ref, b_ref, o_ref):
    bias = jnp.reshape(b_ref[...], (b_ref.shape[0], 1))
    o_ref[...] = (
        jnp.dot(w_ref[...], x_ref[...], preferred_element_type=jnp.float32)
        + bias
    )


def kernel(scalar, W, b):
    batch, k = scalar.shape
    n = W.shape[1]
    xT = scalar.T
    wT = W.T
    grid = batch // BN
    outT = pl.pallas_call(
        _body,
        grid=(grid,),
        in_specs=[
            pl.BlockSpec((k, BN), lambda i: (0, i)),
            pl.BlockSpec((n, k), lambda i: (0, 0)),
            pl.BlockSpec((n,), lambda i: (0,)),
        ],
        out_specs=pl.BlockSpec((n, BN), lambda i: (0, i)),
        out_shape=jax.ShapeDtypeStruct((n, batch), jnp.float32),
    )(xT, wT, b)
    return outT.T
